# baseline (speedup 1.0000x reference)
import functools
import jax
import jax.numpy as jnp
from jax import lax
from jax.experimental import pallas as pl
from jax.experimental.pallas import tpu as pltpu
from jax.experimental.pallas import tpu_sc as plsc

B = 16384
V = 11
D = 64
W = 2 * D
NC, NS = 2, 16
NW = NC * NS
BPW = B // NW
CHUNK = 128
NCH = BPW // CHUNK
TROWS = V * V


@functools.cache
def _make_sc_kernel():
    @functools.partial(
        pl.kernel,
        mesh=plsc.VectorSubcoreMesh(core_axis_name="c", subcore_axis_name="s"),
        out_type=jax.ShapeDtypeStruct((NW * NCH, CHUNK, W), jnp.float32),
        scratch_types=[
            pltpu.VMEM_SHARED((TROWS, W), jnp.float32),
            pltpu.VMEM((V * D,), jnp.float32),
            pltpu.VMEM((V * D,), jnp.float32),
            pltpu.VMEM((TROWS, W), jnp.float32),
            pltpu.VMEM((NCH, CHUNK), jnp.int32),
            pltpu.VMEM((NCH, CHUNK), jnp.int32),
            pltpu.VMEM((NCH, CHUNK), jnp.int32),
            pltpu.VMEM((NCH, CHUNK, W), jnp.float32),
            pltpu.SemaphoreType.DMA,
            pltpu.SemaphoreType.DMA,
            pltpu.SemaphoreType.DMA,
            pltpu.SemaphoreType.DMA,
            pltpu.SemaphoreType.DMA,
            pltpu.SemaphoreType.DMA,
        ],
    )
    def _sc_body(y_hbm, s_hbm, e1_hbm, e2_hbm, out_hbm,
                 tab_sh, e1_v, e2_v, bld_v, y_v, s_v, idx_v, rows_v,
                 sem_in, g0, g1, g2, g3, sem_o):
        gsems = [g0, g1, g2, g3]
        sid = lax.axis_index("s")
        wid = sid * NC + lax.axis_index("c")
        base = wid * NCH
        loads = [
            pltpu.async_copy(y_hbm.at[pl.ds(base, NCH)], y_v, sem_in),
            pltpu.async_copy(s_hbm.at[pl.ds(base, NCH)], s_v, sem_in),
        ]

        @pl.when(sid == 0)
        def _build_and_stage():
            pltpu.sync_copy(e1_hbm, e1_v)
            pltpu.sync_copy(e2_hbm, e2_v)
            for a in range(V):
                cks = [e1_v[pl.ds(a * D + m * 16, 16)] for m in range(D // 16)]
                for b in range(V):
                    r = a * V + b
                    for m in range(D // 16):
                        bld_v[r, pl.ds(m * 16, 16)] = cks[m]
            for b in range(V):
                cks = [e2_v[pl.ds(b * D + m * 16, 16)] for m in range(D // 16)]
                for a in range(V):
                    r = a * V + b
                    for m in range(D // 16):
                        bld_v[r, pl.ds(D + m * 16, 16)] = cks[m]
            pltpu.sync_copy(bld_v, tab_sh)

        for cp in loads:
            cp.wait()
        for c in range(NCH):
            for m in range(CHUNK // 16):
                sl = pl.ds(m * 16, 16)
                idx_v[c, sl] = y_v[c, sl] * V + s_v[c, sl]
        plsc.subcore_barrier()
        gathers = [
            pltpu.async_copy(tab_sh.at[idx_v.at[c]], rows_v.at[c], gsems[c])
            for c in range(NCH)
        ]
        for cp in gathers:
            cp.wait()
        pltpu.sync_copy(rows_v, out_hbm.at[pl.ds(base, NCH)])

    return _sc_body


def kernel(y, s, emb1, emb2):
    y2 = y.astype(jnp.int32).reshape(NW * NCH, CHUNK)
    s2 = s.astype(jnp.int32).reshape(NW * NCH, CHUNK)
    out = _make_sc_kernel()(y2, s2, emb1.reshape(V * D), emb2.reshape(V * D))
    return out.reshape(B, W)


# speedup vs baseline: 1.1965x; 1.1965x over previous
import functools
import jax
import jax.numpy as jnp
from jax import lax
from jax.experimental import pallas as pl
from jax.experimental.pallas import tpu as pltpu
from jax.experimental.pallas import tpu_sc as plsc

B = 16384
V = 11
D = 64
W = 2 * D
NC, NS = 2, 16
NW = NC * NS
BPW = B // NW
CHUNK = 128
NCH = BPW // CHUNK
TROWS = V * V


def _table_body(e1_ref, e2_ref, out_ref):
    for a in range(V):
        out_ref[pl.ds(a * V, V), pl.ds(0, D)] = jnp.broadcast_to(
            e1_ref[pl.ds(a, 1), :], (V, D))
        out_ref[pl.ds(a * V, V), pl.ds(D, D)] = e2_ref[...]


def _build_table(emb1, emb2):
    return pl.pallas_call(
        _table_body,
        out_shape=jax.ShapeDtypeStruct((TROWS, W), jnp.float32),
    )(emb1, emb2)


@functools.cache
def _make_sc_gather():
    @functools.partial(
        pl.kernel,
        mesh=plsc.VectorSubcoreMesh(core_axis_name="c", subcore_axis_name="s"),
        out_type=jax.ShapeDtypeStruct((NW * NCH, CHUNK, W), jnp.float32),
        scratch_types=[
            pltpu.VMEM_SHARED((TROWS, W), jnp.float32),
            pltpu.VMEM((NCH, CHUNK), jnp.int32),
            pltpu.VMEM((NCH, CHUNK), jnp.int32),
            pltpu.VMEM((NCH, CHUNK), jnp.int32),
            pltpu.VMEM((NCH, CHUNK, W), jnp.float32),
            pltpu.SemaphoreType.DMA,
            pltpu.SemaphoreType.DMA,
            pltpu.SemaphoreType.DMA,
            pltpu.SemaphoreType.DMA,
            pltpu.SemaphoreType.DMA,
            pltpu.SemaphoreType.DMA,
        ],
    )
    def _sc_gather(y_hbm, s_hbm, tab_hbm, out_hbm,
                   tab_sh, y_v, s_v, idx_v, rows_v,
                   sem_in, g0, g1, g2, g3, sem_o):
        gsems = [g0, g1, g2, g3]
        sid = lax.axis_index("s")
        wid = sid * NC + lax.axis_index("c")
        base = wid * NCH
        loads = [
            pltpu.async_copy(y_hbm.at[pl.ds(base, NCH)], y_v, sem_in),
            pltpu.async_copy(s_hbm.at[pl.ds(base, NCH)], s_v, sem_in),
        ]

        @pl.when(sid == 0)
        def _stage():
            pltpu.sync_copy(tab_hbm, tab_sh)

        for cp in loads:
            cp.wait()
        for c in range(NCH):
            for m in range(CHUNK // 16):
                sl = pl.ds(m * 16, 16)
                idx_v[c, sl] = y_v[c, sl] * V + s_v[c, sl]
        plsc.subcore_barrier()
        gathers = [
            pltpu.async_copy(tab_sh.at[idx_v.at[c]], rows_v.at[c], gsems[c])
            for c in range(NCH)
        ]
        outs = []
        for c in range(NCH):
            gathers[c].wait()
            outs.append(
                pltpu.async_copy(rows_v.at[c], out_hbm.at[base + c], sem_o))
        for cp in outs:
            cp.wait()

    return _sc_gather


def kernel(y, s, emb1, emb2):
    table = _build_table(emb1, emb2)
    y2 = y.astype(jnp.int32).reshape(NW * NCH, CHUNK)
    s2 = s.astype(jnp.int32).reshape(NW * NCH, CHUNK)
    out = _make_sc_gather()(y2, s2, table)
    return out.reshape(B, W)
